# baseline (speedup 1.0000x reference)
import functools

import jax
import jax.numpy as jnp
from jax import lax
from jax.experimental import pallas as pl
from jax.experimental.pallas import tpu as pltpu
from jax.experimental.pallas import tpu_sc as plsc

B = 16384
EMB = 128
LAT = 64
GATHER = 128
STAGE = 256
NBUF = 3
FA = 2

_NC, _NS = 2, 16
_NW = _NC * _NS
_BPW = B // _NW
_SPT = _BPW // STAGE
_NSTAGE = 3 * _SPT


@functools.cache
def _make_gather3():
    mesh = plsc.VectorSubcoreMesh(
        core_axis_name="c", subcore_axis_name="s", num_cores=_NC
    )

    @functools.partial(
        pl.kernel,
        mesh=mesh,
        out_type=(
            jax.ShapeDtypeStruct((B, EMB), jnp.float32),
            jax.ShapeDtypeStruct((B, EMB), jnp.float32),
            jax.ShapeDtypeStruct((B, EMB), jnp.float32),
        ),
        scratch_types=[
            pltpu.VMEM((_BPW,), jnp.int32),
            pltpu.VMEM((_BPW,), jnp.int32),
            pltpu.VMEM((_BPW,), jnp.int32),
        ]
        + [pltpu.VMEM((STAGE, EMB), jnp.float32) for _ in range(NBUF)]
        + [
            pltpu.SemaphoreType.DMA,
            pltpu.SemaphoreType.DMA,
            pltpu.SemaphoreType.DMA,
        ],
    )
    def gather3(
        g_hbm, b_hbm, p_hbm, gt_hbm, bt_hbm, pt_hbm,
        og_hbm, ob_hbm, op_hbm, ig_v, ib_v, ip_v, *rest,
    ):
        bufs = rest[:NBUF]
        isem, gsem, wsem = rest[NBUF:]
        wid = lax.axis_index("s") * _NC + lax.axis_index("c")
        base = wid * _BPW
        tabs = (gt_hbm, bt_hbm, pt_hbm)
        outs = (og_hbm, ob_hbm, op_hbm)
        idxs = (ig_v, ib_v, ip_v)
        icopies = [
            pltpu.async_copy(idx.at[pl.ds(base, _BPW)], idxs[t], isem)
            for t, idx in enumerate((g_hbm, b_hbm, p_hbm))
        ]
        idx_ready = [False, False, False]

        def fire_gather(s):
            t, h = s // _SPT, s % _SPT
            if not idx_ready[t]:
                icopies[t].wait()
                idx_ready[t] = True
            return [
                pltpu.async_copy(
                    tabs[t].at[idxs[t].at[pl.ds(h * STAGE + j * GATHER, GATHER)]],
                    bufs[s % NBUF].at[pl.ds(j * GATHER, GATHER)],
                    gsem,
                )
                for j in range(STAGE // GATHER)
            ]

        def fire_write(s):
            t, h = s // _SPT, s % _SPT
            return pltpu.async_copy(
                bufs[s % NBUF],
                outs[t].at[pl.ds(base + h * STAGE, STAGE)],
                wsem,
            )

        gathers = {k: fire_gather(k) for k in range(FA)}
        writes = {}
        waited = set()
        for s in range(_NSTAGE):
            for c in gathers[s]:
                c.wait()
            writes[s] = fire_write(s)
            nxt = s + FA
            if nxt < _NSTAGE:
                if nxt >= NBUF:
                    writes[nxt - NBUF].wait()
                    waited.add(nxt - NBUF)
                gathers[nxt] = fire_gather(nxt)
        for s in range(_NSTAGE):
            if s not in waited:
                writes[s].wait()

    return gather3


BLK = 8192


def _proj_body(xg_ref, xb_ref, xp_ref, wp_ref, bp_ref, wf_ref, bf_ref, o_ref):
    h = jnp.dot(xg_ref[...], wp_ref[0:EMB, :], preferred_element_type=jnp.float32)
    h = h + jnp.dot(xb_ref[...], wp_ref[EMB : 2 * EMB, :], preferred_element_type=jnp.float32)
    h = h + jnp.dot(xp_ref[...], wp_ref[2 * EMB : 3 * EMB, :], preferred_element_type=jnp.float32)
    h = h + bp_ref[...]
    ot = lax.dot_general(
        wf_ref[...], h, (((0,), (1,)), ((), ())),
        preferred_element_type=jnp.float32,
    )
    o_ref[...] = ot + bf_ref[...]


def _proj(xg, xb, xp, W_proj, b_proj, W_fc, b_fc):
    return pl.pallas_call(
        _proj_body,
        grid=(B // BLK,),
        in_specs=[
            pl.BlockSpec((BLK, EMB), lambda i: (i, 0)),
            pl.BlockSpec((BLK, EMB), lambda i: (i, 0)),
            pl.BlockSpec((BLK, EMB), lambda i: (i, 0)),
            pl.BlockSpec((3 * EMB, EMB), lambda i: (0, 0)),
            pl.BlockSpec((1, EMB), lambda i: (0, 0)),
            pl.BlockSpec((EMB, LAT), lambda i: (0, 0)),
            pl.BlockSpec((LAT, 1), lambda i: (0, 0)),
        ],
        out_specs=pl.BlockSpec((LAT, BLK), lambda i: (0, i)),
        out_shape=jax.ShapeDtypeStruct((LAT, B), jnp.float32),
    )(xg, xb, xp, W_proj, b_proj.reshape(1, EMB), W_fc, b_fc.reshape(LAT, 1))


def kernel(gains, bws, pms, gain_table, bw_table, pm_table, W_proj, b_proj, W_fc, b_fc):
    g = gains.astype(jnp.int32)
    bw = bws.astype(jnp.int32)
    pm = pms.astype(jnp.int32)
    xg, xb, xp = _make_gather3()(g, bw, pm, gain_table, bw_table, pm_table)
    outT = _proj(xg, xb, xp, W_proj, b_proj, W_fc, b_fc)
    return outT.T
